# Initial kernel scaffold; baseline (speedup 1.0000x reference)
import jax
import jax.numpy as jnp
from jax.experimental import pallas as pl


def kernel(h, edge_index, edge_features, W_e, b_e, W_ih, W_hh, b_ih, b_hh):
    raise NotImplementedError("write your pallas kernel here")


# speedup vs baseline: 1.9942x; 1.9942x over previous
import functools

import jax
import jax.numpy as jnp
from jax import lax
from jax.experimental import pallas as pl
from jax.experimental.pallas import tpu as pltpu
from jax.experimental.pallas import tpu_sc as plsc

_NC = 2
_NS = 16
_NW = _NC * _NS
_C = 128


def _make_gather(N, D, K):
    mesh = plsc.VectorSubcoreMesh(core_axis_name="c", subcore_axis_name="s")

    @functools.partial(
        pl.kernel,
        mesh=mesh,
        out_type=jax.ShapeDtypeStruct((_NW * K * _C, D), jnp.float32),
        scratch_types=[
            pltpu.VMEM((K, _C), jnp.int32),
            pltpu.VMEM((_C, D), jnp.float32),
            pltpu.SemaphoreType.DMA,
        ],
        compiler_params=pltpu.CompilerParams(use_tc_tiling_on_sc=False),
    )
    def gather_k(h_hbm, src_hbm, out_hbm, idx_v, rows_v, sem):
        wid = lax.axis_index("s") * _NC + lax.axis_index("c")
        pltpu.sync_copy(src_hbm.at[wid], idx_v)

        def step(j, carry):
            pltpu.async_copy(h_hbm.at[idx_v.at[j]], rows_v, sem).wait()
            pltpu.sync_copy(rows_v, out_hbm.at[pl.ds((wid * K + j) * _C, _C)])
            return carry

        lax.fori_loop(0, K, step, 0)

    return gather_k


def _make_scatter(N, D, K):
    mesh = plsc.VectorSubcoreMesh(core_axis_name="c", subcore_axis_name="s")
    rows_per_sub = N // _NS

    @functools.partial(
        pl.kernel,
        mesh=mesh,
        out_type=jax.ShapeDtypeStruct((_NC * N, D), jnp.float32),
        scratch_types=[
            pltpu.VMEM((K, _C), jnp.int32),
            pltpu.VMEM((_C, D), jnp.float32),
            pltpu.VMEM_SHARED((N, D), jnp.float32),
            pltpu.SemaphoreType.DMA,
        ],
        compiler_params=pltpu.CompilerParams(use_tc_tiling_on_sc=False),
    )
    def scatter_k(msg_hbm, dst_hbm, zeros_hbm, out_hbm, idx_v, rows_v, acc_sh, sem):
        c = lax.axis_index("c")
        s = lax.axis_index("s")
        wid = s * _NC + c

        base = s * rows_per_sub
        pltpu.sync_copy(zeros_hbm.at[pl.ds(base, rows_per_sub)],
                        acc_sh.at[pl.ds(base, rows_per_sub)])
        plsc.subcore_barrier()

        pltpu.sync_copy(dst_hbm.at[wid], idx_v)

        def step(j, carry):
            pltpu.sync_copy(msg_hbm.at[pl.ds((wid * K + j) * _C, _C)], rows_v)
            pltpu.sync_copy(rows_v, acc_sh.at[idx_v.at[j]], add=True)
            return carry

        lax.fori_loop(0, K, step, 0)
        plsc.subcore_barrier()

        pltpu.sync_copy(acc_sh.at[pl.ds(base, rows_per_sub)],
                        out_hbm.at[pl.ds(c * N + base, rows_per_sub)])

    return scatter_k


def _msg_body(E, ED, D, BE, hw_ref, ef_ref, wt_ref, bt_ref, out_ref):
    hw = hw_ref[...]
    U = jnp.dot(hw, wt_ref[...],
                preferred_element_type=jnp.float32)
    ef = ef_ref[...]
    acc = jnp.dot(hw, bt_ref[...],
                  preferred_element_type=jnp.float32)
    for k in range(ED):
        acc = acc + ef[:, k:k + 1] * U[:, k * D:(k + 1) * D]
    rows = pl.program_id(0) * BE + lax.broadcasted_iota(jnp.int32, (BE, 1), 0)
    out_ref[...] = jnp.where(rows < E, acc, 0.0)


def _gru_body(D, m2_ref, h_ref, wih_ref, whh_ref, bih_ref, bhh_ref, out_ref):
    m = m2_ref[0] + m2_ref[1]
    h = h_ref[...]
    gi = jnp.dot(m, wih_ref[...], preferred_element_type=jnp.float32) + bih_ref[...]
    gh = jnp.dot(h, whh_ref[...], preferred_element_type=jnp.float32) + bhh_ref[...]
    r = jax.nn.sigmoid(gi[:, :D] + gh[:, :D])
    z = jax.nn.sigmoid(gi[:, D:2 * D] + gh[:, D:2 * D])
    n = jnp.tanh(gi[:, 2 * D:] + r * gh[:, 2 * D:])
    out_ref[...] = (1.0 - z) * n + z * h


def kernel(h, edge_index, edge_features, W_e, b_e, W_ih, W_hh, b_ih, b_hh):
    N, D = h.shape
    E, ED = edge_features.shape

    per_chunk = _NW * _C
    K = (E + per_chunk - 1) // per_chunk
    EP = _NW * K * _C

    src = jnp.concatenate(
        [edge_index[0], jnp.zeros((EP - E,), jnp.int32)]).reshape(_NW, K, _C)
    dst = jnp.concatenate(
        [edge_index[1], jnp.zeros((EP - E,), jnp.int32)]).reshape(_NW, K, _C)
    ef = jnp.concatenate(
        [edge_features, jnp.zeros((EP - E, ED), jnp.float32)], axis=0)

    h_w = _make_gather(N, D, K)(h, src)

    Wt = W_e.reshape(ED, D, D).transpose(2, 0, 1).reshape(D, ED * D)
    bt = b_e.reshape(D, D).T
    BE = 2048
    msg = pl.pallas_call(
        functools.partial(_msg_body, E, ED, D, BE),
        grid=(EP // BE,),
        in_specs=[
            pl.BlockSpec((BE, D), lambda i: (i, 0)),
            pl.BlockSpec((BE, ED), lambda i: (i, 0)),
            pl.BlockSpec((D, ED * D), lambda i: (0, 0)),
            pl.BlockSpec((D, D), lambda i: (0, 0)),
        ],
        out_specs=pl.BlockSpec((BE, D), lambda i: (i, 0)),
        out_shape=jax.ShapeDtypeStruct((EP, D), jnp.float32),
    )(h_w, ef, Wt, bt)

    zeros = jnp.zeros((N, D), jnp.float32)
    m2 = _make_scatter(N, D, K)(msg, dst, zeros)
    m2 = m2.reshape(_NC, N, D)

    BN = 2000
    h_new = pl.pallas_call(
        functools.partial(_gru_body, D),
        grid=(N // BN,),
        in_specs=[
            pl.BlockSpec((_NC, BN, D), lambda i: (0, i, 0)),
            pl.BlockSpec((BN, D), lambda i: (i, 0)),
            pl.BlockSpec((D, 3 * D), lambda i: (0, 0)),
            pl.BlockSpec((D, 3 * D), lambda i: (0, 0)),
            pl.BlockSpec((1, 3 * D), lambda i: (0, 0)),
            pl.BlockSpec((1, 3 * D), lambda i: (0, 0)),
        ],
        out_specs=pl.BlockSpec((BN, D), lambda i: (i, 0)),
        out_shape=jax.ShapeDtypeStruct((N, D), jnp.float32),
    )(m2, h, W_ih.T, W_hh.T, b_ih.reshape(1, 3 * D), b_hh.reshape(1, 3 * D))

    return h_new


# speedup vs baseline: 3.7848x; 1.8979x over previous
import functools

import jax
import jax.numpy as jnp
import numpy as np
from jax import lax
from jax.experimental import pallas as pl
from jax.experimental.pallas import tpu as pltpu
from jax.experimental.pallas import tpu_sc as plsc

_NC = 2
_NS = 16
_NW = _NC * _NS
_C = 128


def _make_gather(N, D, K):
    mesh = plsc.VectorSubcoreMesh(core_axis_name="c", subcore_axis_name="s")

    @functools.partial(
        pl.kernel,
        mesh=mesh,
        out_type=jax.ShapeDtypeStruct((_NW * K * _C, D), jnp.float32),
        scratch_types=[
            pltpu.VMEM((K, _C), jnp.int32),
            pltpu.VMEM((_C, D), jnp.float32),
            pltpu.SemaphoreType.DMA,
        ],
        compiler_params=pltpu.CompilerParams(use_tc_tiling_on_sc=False),
    )
    def gather_k(h_hbm, src_hbm, out_hbm, idx_v, rows_v, sem):
        wid = lax.axis_index("s") * _NC + lax.axis_index("c")
        pltpu.sync_copy(src_hbm.at[wid], idx_v)

        def step(j, carry):
            pltpu.async_copy(h_hbm.at[idx_v.at[j]], rows_v, sem).wait()
            pltpu.sync_copy(rows_v, out_hbm.at[pl.ds((wid * K + j) * _C, _C)])
            return carry

        lax.fori_loop(0, K, step, 0)

    return gather_k


def _make_scatter(N, D, K):
    mesh = plsc.VectorSubcoreMesh(core_axis_name="c", subcore_axis_name="s")
    rows_per_sub = N // _NS

    @functools.partial(
        pl.kernel,
        mesh=mesh,
        out_type=jax.ShapeDtypeStruct((_NC * N, D), jnp.float32),
        scratch_types=[
            pltpu.VMEM((K, _C), jnp.int32),
            pltpu.VMEM((_C, D), jnp.float32),
            pltpu.VMEM_SHARED((N, D), jnp.float32),
            pltpu.SemaphoreType.DMA,
        ],
        compiler_params=pltpu.CompilerParams(use_tc_tiling_on_sc=False),
    )
    def scatter_k(msg_hbm, dst_hbm, zeros_hbm, out_hbm, idx_v, rows_v, acc_sh, sem):
        c = lax.axis_index("c")
        s = lax.axis_index("s")
        wid = s * _NC + c

        base = s * rows_per_sub
        pltpu.sync_copy(zeros_hbm.at[pl.ds(base, rows_per_sub)],
                        acc_sh.at[pl.ds(base, rows_per_sub)])
        plsc.subcore_barrier()

        pltpu.sync_copy(dst_hbm.at[wid], idx_v)

        def step(j, carry):
            pltpu.sync_copy(msg_hbm.at[pl.ds((wid * K + j) * _C, _C)], rows_v)
            pltpu.sync_copy(rows_v, acc_sh.at[idx_v.at[j]], add=True)
            return carry

        lax.fori_loop(0, K, step, 0)
        plsc.subcore_barrier()

        pltpu.sync_copy(acc_sh.at[pl.ds(base, rows_per_sub)],
                        out_hbm.at[pl.ds(c * N + base, rows_per_sub)])

    return scatter_k


def _msg_body(E, ED, D, BE, hw_ref, ef_ref, wt_ref, bt_ref, s_ref, r_ref,
              out_ref):
    hw = hw_ref[...]
    V = jnp.dot(hw, wt_ref[...],
                preferred_element_type=jnp.float32)
    EQ = jnp.dot(ef_ref[...], s_ref[...],
                 preferred_element_type=jnp.float32)
    P = V * EQ
    acc = jnp.dot(P, r_ref[...],
                  preferred_element_type=jnp.float32)
    acc = acc + jnp.dot(hw, bt_ref[...],
                        preferred_element_type=jnp.float32)
    rows = pl.program_id(0) * BE + lax.broadcasted_iota(jnp.int32, (BE, 1), 0)
    out_ref[...] = jnp.where(rows < E, acc, 0.0)


def _gru_body(D, m2_ref, h_ref, wih_ref, whh_ref, bih_ref, bhh_ref, out_ref):
    m = m2_ref[0] + m2_ref[1]
    h = h_ref[...]
    gi = jnp.dot(m, wih_ref[...], preferred_element_type=jnp.float32) + bih_ref[...]
    gh = jnp.dot(h, whh_ref[...], preferred_element_type=jnp.float32) + bhh_ref[...]
    r = jax.nn.sigmoid(gi[:, :D] + gh[:, :D])
    z = jax.nn.sigmoid(gi[:, D:2 * D] + gh[:, D:2 * D])
    n = jnp.tanh(gi[:, 2 * D:] + r * gh[:, 2 * D:])
    out_ref[...] = (1.0 - z) * n + z * h


def kernel(h, edge_index, edge_features, W_e, b_e, W_ih, W_hh, b_ih, b_hh):
    N, D = h.shape
    E, ED = edge_features.shape

    per_chunk = _NW * _C
    K = (E + per_chunk - 1) // per_chunk
    EP = _NW * K * _C

    src = jnp.concatenate(
        [edge_index[0], jnp.zeros((EP - E,), jnp.int32)]).reshape(_NW, K, _C)
    dst = jnp.concatenate(
        [edge_index[1], jnp.zeros((EP - E,), jnp.int32)]).reshape(_NW, K, _C)
    ef = jnp.concatenate(
        [edge_features, jnp.zeros((EP - E, ED), jnp.float32)], axis=0)

    h_w = _make_gather(N, D, K)(h, src)

    Wt = W_e.reshape(ED, D, D).transpose(2, 0, 1).reshape(D, ED * D)
    bt = b_e.reshape(D, D).T
    S = jnp.asarray(np.repeat(np.eye(ED, dtype=np.float32), D, axis=1))
    R = jnp.asarray(np.tile(np.eye(D, dtype=np.float32), (ED, 1)))
    BE = 2048
    msg = pl.pallas_call(
        functools.partial(_msg_body, E, ED, D, BE),
        grid=(EP // BE,),
        in_specs=[
            pl.BlockSpec((BE, D), lambda i: (i, 0)),
            pl.BlockSpec((BE, ED), lambda i: (i, 0)),
            pl.BlockSpec((D, ED * D), lambda i: (0, 0)),
            pl.BlockSpec((D, D), lambda i: (0, 0)),
            pl.BlockSpec((ED, ED * D), lambda i: (0, 0)),
            pl.BlockSpec((ED * D, D), lambda i: (0, 0)),
        ],
        out_specs=pl.BlockSpec((BE, D), lambda i: (i, 0)),
        out_shape=jax.ShapeDtypeStruct((EP, D), jnp.float32),
    )(h_w, ef, Wt, bt, S, R)

    zeros = jnp.zeros((N, D), jnp.float32)
    m2 = _make_scatter(N, D, K)(msg, dst, zeros)
    m2 = m2.reshape(_NC, N, D)

    BN = 2000
    h_new = pl.pallas_call(
        functools.partial(_gru_body, D),
        grid=(N // BN,),
        in_specs=[
            pl.BlockSpec((_NC, BN, D), lambda i: (0, i, 0)),
            pl.BlockSpec((BN, D), lambda i: (i, 0)),
            pl.BlockSpec((D, 3 * D), lambda i: (0, 0)),
            pl.BlockSpec((D, 3 * D), lambda i: (0, 0)),
            pl.BlockSpec((1, 3 * D), lambda i: (0, 0)),
            pl.BlockSpec((1, 3 * D), lambda i: (0, 0)),
        ],
        out_specs=pl.BlockSpec((BN, D), lambda i: (i, 0)),
        out_shape=jax.ShapeDtypeStruct((N, D), jnp.float32),
    )(m2, h, W_ih.T, W_hh.T, b_ih.reshape(1, 3 * D), b_hh.reshape(1, 3 * D))

    return h_new
